# baseline (speedup 1.0000x reference)
import functools

import jax
import jax.numpy as jnp
from jax import lax
from jax.experimental import pallas as pl
from jax.experimental.pallas import tpu as pltpu
from jax.experimental.pallas import tpu_sc as plsc

_NUM_WORKERS = 32
_CHUNK = 80
_LANES = 16


def _cross_and_degrees(feat_bf, norms, idx_a, idx_b):
    n_nodes, dw = feat_bf.shape
    n_edges = idx_a.shape[0]
    per_w = n_edges // _NUM_WORKERS
    n_chunks = per_w // _CHUNK
    assert n_chunks % 2 == 1, "pipeline structure expects an odd chunk count"
    assert n_nodes % _LANES == 0
    mesh = plsc.VectorSubcoreMesh(core_axis_name="c", subcore_axis_name="s")

    @functools.partial(
        pl.kernel,
        mesh=mesh,
        compiler_params=pltpu.CompilerParams(needs_layout_passes=False, use_tc_tiling_on_sc=False),
        out_type=jax.ShapeDtypeStruct((_NUM_WORKERS, _LANES), jnp.float32),
        scratch_types=[
            pltpu.VMEM((per_w,), jnp.int32),
            pltpu.VMEM((per_w,), jnp.int32),
            pltpu.VMEM((_CHUNK, dw), jnp.int32),
            pltpu.VMEM((_CHUNK, dw), jnp.int32),
            pltpu.VMEM((_CHUNK, dw), jnp.int32),
            pltpu.VMEM((_CHUNK, dw), jnp.int32),
            pltpu.VMEM((n_nodes,), jnp.int32),
            pltpu.VMEM((n_nodes,), jnp.float32),
            pltpu.VMEM((_LANES,), jnp.float32),
            pltpu.VMEM_SHARED((n_nodes, dw), jnp.int32),
            pltpu.SemaphoreType.DMA,
            pltpu.SemaphoreType.DMA,
            pltpu.SemaphoreType.DMA,
            pltpu.SemaphoreType.DMA,
            pltpu.SemaphoreType.DMA,
            pltpu.SemaphoreType.DMA,
            pltpu.SemaphoreType.DMA,
            pltpu.SemaphoreType.DMA,
        ],
    )
    def lap_kernel(feat_hbm, norms_hbm, ia_hbm, ib_hbm, out_hbm,
                   ia_v, ib_v, ra0, rb0, ra1, rb1, hist_v, xn_v, res_v,
                   feat_sh, sa0, sb0, sa1, sb1, ta0, tb0, ta1, tb1):
        wid = lax.axis_index("s") * 2 + lax.axis_index("c")
        base = wid * per_w
        sid = lax.axis_index("s")
        stripe = n_nodes // 16
        pltpu.sync_copy(feat_hbm.at[pl.ds(sid * stripe, stripe)],
                        feat_sh.at[pl.ds(sid * stripe, stripe)])
        pltpu.sync_copy(ia_hbm.at[pl.ds(base, per_w)], ia_v)
        pltpu.sync_copy(ib_hbm.at[pl.ds(base, per_w)], ib_v)
        pltpu.sync_copy(norms_hbm, xn_v)
        plsc.subcore_barrier()

        _H = _CHUNK // 2

        def issue(ci, buf_a, buf_b, sem_a, sem_b, sem_a2, sem_b2):
            off = ci * _CHUNK
            pltpu.async_copy(feat_sh.at[ia_v.at[pl.ds(off, _H)]],
                             buf_a.at[pl.ds(0, _H)], sem_a)
            pltpu.async_copy(feat_sh.at[ib_v.at[pl.ds(off, _H)]],
                             buf_b.at[pl.ds(0, _H)], sem_b)
            pltpu.async_copy(feat_sh.at[ia_v.at[pl.ds(off + _H, _H)]],
                             buf_a.at[pl.ds(_H, _H)], sem_a2)
            pltpu.async_copy(feat_sh.at[ib_v.at[pl.ds(off + _H, _H)]],
                             buf_b.at[pl.ds(_H, _H)], sem_b2)

        def wait(buf_a, buf_b, sem_a, sem_b, sem_a2, sem_b2):
            src = feat_sh.at[pl.ds(0, _H)]
            pltpu.make_async_copy(src, buf_a.at[pl.ds(0, _H)], sem_a).wait()
            pltpu.make_async_copy(src, buf_b.at[pl.ds(0, _H)], sem_b).wait()
            pltpu.make_async_copy(src, buf_a.at[pl.ds(_H, _H)], sem_a2).wait()
            pltpu.make_async_copy(src, buf_b.at[pl.ds(_H, _H)], sem_b2).wait()

        zeros_i = jnp.zeros((_LANES,), jnp.int32)
        ones_i = jnp.ones((_LANES,), jnp.int32)

        def zero_body(j, _):
            hist_v[pl.ds(j * _LANES, _LANES)] = zeros_i
            return 0

        def hist_body(j, _):
            plsc.addupdate_scatter(hist_v, [ia_v[pl.ds(j * _LANES, _LANES)]],
                                   ones_i)
            plsc.addupdate_scatter(hist_v, [ib_v[pl.ds(j * _LANES, _LANES)]],
                                   ones_i)
            return 0

        issue(0, ra0, rb0, sa0, sb0, ta0, tb0)
        lax.fori_loop(0, n_nodes // _LANES, zero_body, 0)
        lax.fori_loop(0, per_w // _LANES, hist_body, 0)

        def compute(ba, bb, acc):
            def edge_body(ei, accs):
                r0, r1 = accs
                for k in range(dw // _LANES):
                    va = ba[ei, pl.ds(k * _LANES, _LANES)]
                    vb = bb[ei, pl.ds(k * _LANES, _LANES)]
                    la = lax.bitcast_convert_type(va << 16, jnp.float32)
                    lb = lax.bitcast_convert_type(vb << 16, jnp.float32)
                    ha = lax.bitcast_convert_type(va, jnp.float32)
                    hb = lax.bitcast_convert_type(vb, jnp.float32)
                    r0 = r0 + la * lb
                    r1 = r1 + ha * hb
                return (r0, r1)

            return lax.fori_loop(0, _CHUNK, edge_body, acc)

        def pair_body(i, acc):
            issue(2 * i + 1, ra1, rb1, sa1, sb1, ta1, tb1)
            wait(ra0, rb0, sa0, sb0, ta0, tb0)
            acc = compute(ra0, rb0, acc)
            issue(2 * i + 2, ra0, rb0, sa0, sb0, ta0, tb0)
            wait(ra1, rb1, sa1, sb1, ta1, tb1)
            return compute(ra1, rb1, acc)

        zero_f = jnp.zeros((_LANES,), jnp.float32)
        acc = lax.fori_loop(0, (n_chunks - 1) // 2, pair_body,
                            (zero_f, zero_f))
        wait(ra0, rb0, sa0, sb0, ta0, tb0)
        r0, r1 = compute(ra0, rb0, acc)

        def deg_body(j, accn):
            h = hist_v[pl.ds(j * _LANES, _LANES)].astype(jnp.float32)
            x = xn_v[pl.ds(j * _LANES, _LANES)]
            return accn + h * x

        rn = lax.fori_loop(0, n_nodes // _LANES, deg_body,
                           jnp.zeros((_LANES,), jnp.float32))
        res_v[...] = rn - 2.0 * (r0 + r1)
        pltpu.sync_copy(res_v, out_hbm.at[wid])

    return lap_kernel(feat_bf, norms, idx_a, idx_b)


def kernel(features, indices):
    n_edges = indices.shape[1]
    n_nodes, d = features.shape
    dw = d // 2
    fi = lax.bitcast_convert_type(features, jnp.int32)
    feat_packed = (fi[:, 1::2] & jnp.int32(-65536)) | lax.shift_right_logical(
        fi[:, 0::2], 16)
    norms = jnp.sum(features * features, axis=1)
    partials = _cross_and_degrees(feat_packed, norms, indices[0], indices[1])
    return jnp.sum(partials) / n_edges


# speedup vs baseline: 2.5864x; 2.5864x over previous
import functools

import jax
import jax.numpy as jnp
from jax import lax
from jax.experimental import pallas as pl
from jax.experimental.pallas import tpu as pltpu
from jax.experimental.pallas import tpu_sc as plsc

_NUM_WORKERS = 32
_CHUNK = 80
_LANES = 16


def _cross_and_degrees(feat_bf, norms, idx_a, idx_b):
    n_nodes, dw = feat_bf.shape
    n_edges = idx_a.shape[0]
    per_w = n_edges // _NUM_WORKERS
    n_chunks = per_w // _CHUNK
    assert n_chunks % 2 == 1, "pipeline structure expects an odd chunk count"
    assert n_nodes % _LANES == 0
    mesh = plsc.VectorSubcoreMesh(core_axis_name="c", subcore_axis_name="s")

    @functools.partial(
        pl.kernel,
        mesh=mesh,
        compiler_params=pltpu.CompilerParams(needs_layout_passes=False, use_tc_tiling_on_sc=False),
        out_type=jax.ShapeDtypeStruct((_NUM_WORKERS, _LANES), jnp.float32),
        scratch_types=[
            pltpu.VMEM((per_w,), jnp.int32),
            pltpu.VMEM((per_w,), jnp.int32),
            pltpu.VMEM((_CHUNK, dw), jnp.int32),
            pltpu.VMEM((_CHUNK, dw), jnp.int32),
            pltpu.VMEM((_CHUNK, dw), jnp.int32),
            pltpu.VMEM((_CHUNK, dw), jnp.int32),
            pltpu.VMEM((n_nodes,), jnp.int32),
            pltpu.VMEM((n_nodes,), jnp.float32),
            pltpu.VMEM((_LANES,), jnp.float32),
            pltpu.VMEM_SHARED((n_nodes, dw), jnp.int32),
            pltpu.SemaphoreType.DMA,
            pltpu.SemaphoreType.DMA,
            pltpu.SemaphoreType.DMA,
            pltpu.SemaphoreType.DMA,
            pltpu.SemaphoreType.DMA,
            pltpu.SemaphoreType.DMA,
            pltpu.SemaphoreType.DMA,
            pltpu.SemaphoreType.DMA,
        ],
    )
    def lap_kernel(feat_hbm, norms_hbm, ia_hbm, ib_hbm, out_hbm,
                   ia_v, ib_v, ra0, rb0, ra1, rb1, hist_v, xn_v, res_v,
                   feat_sh, sa0, sb0, sa1, sb1, ta0, tb0, ta1, tb1):
        wid = lax.axis_index("s") * 2 + lax.axis_index("c")
        base = wid * per_w
        sid = lax.axis_index("s")
        stripe = n_nodes // 16
        pltpu.sync_copy(feat_hbm.at[pl.ds(sid * stripe, stripe)],
                        feat_sh.at[pl.ds(sid * stripe, stripe)])
        pltpu.sync_copy(ia_hbm.at[pl.ds(base, per_w)], ia_v)
        pltpu.sync_copy(ib_hbm.at[pl.ds(base, per_w)], ib_v)
        pltpu.sync_copy(norms_hbm, xn_v)
        plsc.subcore_barrier()

        _H = _CHUNK // 2

        def issue(ci, buf_a, buf_b, sem_a, sem_b, sem_a2, sem_b2):
            off = ci * _CHUNK
            pltpu.async_copy(feat_sh.at[ia_v.at[pl.ds(off, _H)]],
                             buf_a.at[pl.ds(0, _H)], sem_a)
            pltpu.async_copy(feat_sh.at[ib_v.at[pl.ds(off, _H)]],
                             buf_b.at[pl.ds(0, _H)], sem_b)
            pltpu.async_copy(feat_sh.at[ia_v.at[pl.ds(off + _H, _H)]],
                             buf_a.at[pl.ds(_H, _H)], sem_a2)
            pltpu.async_copy(feat_sh.at[ib_v.at[pl.ds(off + _H, _H)]],
                             buf_b.at[pl.ds(_H, _H)], sem_b2)

        def wait(buf_a, buf_b, sem_a, sem_b, sem_a2, sem_b2):
            src = feat_sh.at[pl.ds(0, _H)]
            pltpu.make_async_copy(src, buf_a.at[pl.ds(0, _H)], sem_a).wait()
            pltpu.make_async_copy(src, buf_b.at[pl.ds(0, _H)], sem_b).wait()
            pltpu.make_async_copy(src, buf_a.at[pl.ds(_H, _H)], sem_a2).wait()
            pltpu.make_async_copy(src, buf_b.at[pl.ds(_H, _H)], sem_b2).wait()

        zeros_i = jnp.zeros((_LANES,), jnp.int32)
        ones_i = jnp.ones((_LANES,), jnp.int32)

        def zero_body(j, _):
            hist_v[pl.ds(j * _LANES, _LANES)] = zeros_i
            return 0

        def hist_body(j, _):
            plsc.addupdate_scatter(hist_v, [ia_v[pl.ds(j * _LANES, _LANES)]],
                                   ones_i)
            plsc.addupdate_scatter(hist_v, [ib_v[pl.ds(j * _LANES, _LANES)]],
                                   ones_i)
            return 0

        issue(0, ra0, rb0, sa0, sb0, ta0, tb0)
        lax.fori_loop(0, n_nodes // _LANES, zero_body, 0)
        lax.fori_loop(0, per_w // _LANES, hist_body, 0)

        def compute(ba, bb, acc):
            def edge_body(ei, accs):
                r0, r1 = accs
                for k in range(dw // _LANES):
                    va = ba[ei, pl.ds(k * _LANES, _LANES)]
                    vb = bb[ei, pl.ds(k * _LANES, _LANES)]
                    la = lax.bitcast_convert_type(va << 16, jnp.float32)
                    lb = lax.bitcast_convert_type(vb << 16, jnp.float32)
                    ha = lax.bitcast_convert_type(va, jnp.float32)
                    hb = lax.bitcast_convert_type(vb, jnp.float32)
                    r0 = r0 + la * lb
                    r1 = r1 + ha * hb
                return (r0, r1)

            return lax.fori_loop(0, _CHUNK, edge_body, acc)

        def pair_body(i, acc):
            issue(2 * i + 1, ra1, rb1, sa1, sb1, ta1, tb1)
            wait(ra0, rb0, sa0, sb0, ta0, tb0)
            acc = compute(ra0, rb0, acc)
            issue(2 * i + 2, ra0, rb0, sa0, sb0, ta0, tb0)
            wait(ra1, rb1, sa1, sb1, ta1, tb1)
            return compute(ra1, rb1, acc)

        zero_f = jnp.zeros((_LANES,), jnp.float32)
        acc = lax.fori_loop(0, (n_chunks - 1) // 2, pair_body,
                            (zero_f, zero_f))
        wait(ra0, rb0, sa0, sb0, ta0, tb0)
        r0, r1 = compute(ra0, rb0, acc)

        def deg_body(j, accn):
            h = hist_v[pl.ds(j * _LANES, _LANES)].astype(jnp.float32)
            x = xn_v[pl.ds(j * _LANES, _LANES)]
            return accn + h * x

        rn = lax.fori_loop(0, n_nodes // _LANES, deg_body,
                           jnp.zeros((_LANES,), jnp.float32))
        res_v[...] = rn - 2.0 * (r0 + r1)
        pltpu.sync_copy(res_v, out_hbm.at[wid])

    return lap_kernel(feat_bf, norms, idx_a, idx_b)


def kernel(features, indices):
    n_edges = indices.shape[1]
    n_nodes, d = features.shape
    dw = d // 2
    fi = lax.bitcast_convert_type(features, jnp.int32)
    feat_packed = (fi[:, :dw] & jnp.int32(-65536)) | lax.shift_right_logical(
        fi[:, dw:], 16)
    norms = jnp.sum(features * features, axis=1)
    partials = _cross_and_degrees(feat_packed, norms, indices[0], indices[1])
    return jnp.sum(partials) / n_edges


# speedup vs baseline: 2.8205x; 1.0905x over previous
import functools

import jax
import jax.numpy as jnp
from jax import lax
from jax.experimental import pallas as pl
from jax.experimental.pallas import tpu as pltpu
from jax.experimental.pallas import tpu_sc as plsc

_NUM_WORKERS = 32
_CHUNK = 80
_LANES = 16


def _cross_and_degrees(feat_bf, norms, indices):
    n_nodes, dw = feat_bf.shape
    n_edges = indices.shape[1]
    per_w = n_edges // _NUM_WORKERS
    n_chunks = per_w // _CHUNK
    assert n_chunks % 2 == 1, "pipeline structure expects an odd chunk count"
    assert n_nodes % _LANES == 0
    mesh = plsc.VectorSubcoreMesh(core_axis_name="c", subcore_axis_name="s")

    @functools.partial(
        pl.kernel,
        mesh=mesh,
        compiler_params=pltpu.CompilerParams(needs_layout_passes=False, use_tc_tiling_on_sc=False),
        out_type=jax.ShapeDtypeStruct((_NUM_WORKERS, _LANES), jnp.float32),
        scratch_types=[
            pltpu.VMEM((per_w,), jnp.int32),
            pltpu.VMEM((per_w,), jnp.int32),
            pltpu.VMEM((_CHUNK, dw), jnp.int32),
            pltpu.VMEM((_CHUNK, dw), jnp.int32),
            pltpu.VMEM((_CHUNK, dw), jnp.int32),
            pltpu.VMEM((_CHUNK, dw), jnp.int32),
            pltpu.VMEM((n_nodes,), jnp.int32),
            pltpu.VMEM((n_nodes,), jnp.float32),
            pltpu.VMEM((_LANES,), jnp.float32),
            pltpu.VMEM_SHARED((n_nodes, dw), jnp.int32),
            pltpu.SemaphoreType.DMA,
            pltpu.SemaphoreType.DMA,
            pltpu.SemaphoreType.DMA,
            pltpu.SemaphoreType.DMA,
            pltpu.SemaphoreType.DMA,
            pltpu.SemaphoreType.DMA,
            pltpu.SemaphoreType.DMA,
            pltpu.SemaphoreType.DMA,
        ],
    )
    def lap_kernel(feat_hbm, norms_hbm, idx_hbm, out_hbm,
                   ia_v, ib_v, ra0, rb0, ra1, rb1, hist_v, xn_v, res_v,
                   feat_sh, sa0, sb0, sa1, sb1, ta0, tb0, ta1, tb1):
        wid = lax.axis_index("s") * 2 + lax.axis_index("c")
        base = wid * per_w
        sid = lax.axis_index("s")
        stripe = n_nodes // 16
        pltpu.sync_copy(feat_hbm.at[pl.ds(sid * stripe, stripe)],
                        feat_sh.at[pl.ds(sid * stripe, stripe)])
        pltpu.sync_copy(idx_hbm.at[0, pl.ds(base, per_w)], ia_v)
        pltpu.sync_copy(idx_hbm.at[1, pl.ds(base, per_w)], ib_v)
        pltpu.sync_copy(norms_hbm, xn_v)
        plsc.subcore_barrier()

        _H = _CHUNK // 2

        def issue(ci, buf_a, buf_b, sem_a, sem_b, sem_a2, sem_b2):
            off = ci * _CHUNK
            pltpu.async_copy(feat_sh.at[ia_v.at[pl.ds(off, _H)]],
                             buf_a.at[pl.ds(0, _H)], sem_a)
            pltpu.async_copy(feat_sh.at[ib_v.at[pl.ds(off, _H)]],
                             buf_b.at[pl.ds(0, _H)], sem_b)
            pltpu.async_copy(feat_sh.at[ia_v.at[pl.ds(off + _H, _H)]],
                             buf_a.at[pl.ds(_H, _H)], sem_a2)
            pltpu.async_copy(feat_sh.at[ib_v.at[pl.ds(off + _H, _H)]],
                             buf_b.at[pl.ds(_H, _H)], sem_b2)

        def wait(buf_a, buf_b, sem_a, sem_b, sem_a2, sem_b2):
            src = feat_sh.at[pl.ds(0, _H)]
            pltpu.make_async_copy(src, buf_a.at[pl.ds(0, _H)], sem_a).wait()
            pltpu.make_async_copy(src, buf_b.at[pl.ds(0, _H)], sem_b).wait()
            pltpu.make_async_copy(src, buf_a.at[pl.ds(_H, _H)], sem_a2).wait()
            pltpu.make_async_copy(src, buf_b.at[pl.ds(_H, _H)], sem_b2).wait()

        zeros_i = jnp.zeros((_LANES,), jnp.int32)
        ones_i = jnp.ones((_LANES,), jnp.int32)

        def zero_body(j, _):
            hist_v[pl.ds(j * _LANES, _LANES)] = zeros_i
            return 0

        def hist_body(j, _):
            plsc.addupdate_scatter(hist_v, [ia_v[pl.ds(j * _LANES, _LANES)]],
                                   ones_i)
            plsc.addupdate_scatter(hist_v, [ib_v[pl.ds(j * _LANES, _LANES)]],
                                   ones_i)
            return 0

        issue(0, ra0, rb0, sa0, sb0, ta0, tb0)
        lax.fori_loop(0, n_nodes // _LANES, zero_body, 0)
        lax.fori_loop(0, per_w // _LANES, hist_body, 0)

        def compute(ba, bb, acc):
            def edge_body(ei, accs):
                r0, r1 = accs
                for k in range(dw // _LANES):
                    va = ba[ei, pl.ds(k * _LANES, _LANES)]
                    vb = bb[ei, pl.ds(k * _LANES, _LANES)]
                    la = lax.bitcast_convert_type(va << 16, jnp.float32)
                    lb = lax.bitcast_convert_type(vb << 16, jnp.float32)
                    ha = lax.bitcast_convert_type(va, jnp.float32)
                    hb = lax.bitcast_convert_type(vb, jnp.float32)
                    r0 = r0 + la * lb
                    r1 = r1 + ha * hb
                return (r0, r1)

            return lax.fori_loop(0, _CHUNK, edge_body, acc)

        def pair_body(i, acc):
            issue(2 * i + 1, ra1, rb1, sa1, sb1, ta1, tb1)
            wait(ra0, rb0, sa0, sb0, ta0, tb0)
            acc = compute(ra0, rb0, acc)
            issue(2 * i + 2, ra0, rb0, sa0, sb0, ta0, tb0)
            wait(ra1, rb1, sa1, sb1, ta1, tb1)
            return compute(ra1, rb1, acc)

        zero_f = jnp.zeros((_LANES,), jnp.float32)
        acc = lax.fori_loop(0, (n_chunks - 1) // 2, pair_body,
                            (zero_f, zero_f))
        wait(ra0, rb0, sa0, sb0, ta0, tb0)
        r0, r1 = compute(ra0, rb0, acc)

        def deg_body(j, accn):
            h = hist_v[pl.ds(j * _LANES, _LANES)].astype(jnp.float32)
            x = xn_v[pl.ds(j * _LANES, _LANES)]
            return accn + h * x

        rn = lax.fori_loop(0, n_nodes // _LANES, deg_body,
                           jnp.zeros((_LANES,), jnp.float32))
        res_v[...] = rn - 2.0 * (r0 + r1)
        pltpu.sync_copy(res_v, out_hbm.at[wid])

    return lap_kernel(feat_bf, norms, indices)


def kernel(features, indices):
    n_edges = indices.shape[1]
    n_nodes, d = features.shape
    dw = d // 2
    fi = lax.bitcast_convert_type(features, jnp.int32)
    feat_packed = (fi[:, :dw] & jnp.int32(-65536)) | lax.shift_right_logical(
        fi[:, dw:], 16)
    norms = jnp.sum(features * features, axis=1)
    partials = _cross_and_degrees(feat_packed, norms, indices)
    return jnp.sum(partials) / n_edges
